# Initial kernel scaffold; baseline (speedup 1.0000x reference)
import jax
import jax.numpy as jnp
from jax.experimental import pallas as pl


def kernel(x, table):
    raise NotImplementedError("write your pallas kernel here")


# speedup vs baseline: 7.9651x; 7.9651x over previous
import functools
import math

import jax
import jax.numpy as jnp
from jax import lax
from jax.experimental import pallas as pl
from jax.experimental.pallas import tpu as pltpu
from jax.experimental.pallas import tpu_sc as plsc

D_EMB = 128
SCALE = math.sqrt(float(D_EMB))

_info = plsc.get_sparse_core_info()
_NC = _info.num_cores
_NS = _info.num_subcores
_NW = _NC * _NS

_CH = 128
_NBUF = 4


def _scale_body(t_ref, o_ref):
    o_ref[...] = t_ref[...] * SCALE


def _scale_table(table):
    v, d = table.shape
    blk = 2000
    assert v % blk == 0
    return pl.pallas_call(
        _scale_body,
        grid=(v // blk,),
        in_specs=[pl.BlockSpec((blk, d), lambda i: (i, 0))],
        out_specs=pl.BlockSpec((blk, d), lambda i: (i, 0)),
        out_shape=jax.ShapeDtypeStruct((v, d), table.dtype),
    )(table)


@functools.partial(jax.jit, static_argnames=("nch",))
def _gather(idx, table, nch):
    b_total = _NW * nch * _CH
    mesh = plsc.VectorSubcoreMesh(core_axis_name="c", subcore_axis_name="s")

    @functools.partial(
        pl.kernel,
        mesh=mesh,
        out_type=jax.ShapeDtypeStruct((b_total, D_EMB), jnp.float32),
        scratch_types=[
            pltpu.VMEM((nch, _CH), jnp.int32),
            pltpu.VMEM((_NBUF, _CH, D_EMB), jnp.float32),
            pltpu.SemaphoreType.DMA((_NBUF,)),
        ],
    )
    def k(idx_hbm, table_hbm, out_hbm, idx_v, rows_v, gsem):
        wid = lax.axis_index("s") * _NC + lax.axis_index("c")
        base = wid * (nch * _CH)
        pltpu.sync_copy(idx_hbm.at[wid], idx_v)

        def start_gather(j, b):
            pltpu.async_copy(table_hbm.at[idx_v.at[j]], rows_v.at[b], gsem.at[b])

        def wait_gather(j, b):
            pltpu.make_async_copy(
                table_hbm.at[idx_v.at[j]], rows_v.at[b], gsem.at[b]
            ).wait()

        def scatter(j, b):
            pltpu.sync_copy(rows_v.at[b], out_hbm.at[pl.ds(base + j * _CH, _CH)])

        for b in range(_NBUF):
            start_gather(b, b)

        def group(g, carry):
            for b in range(_NBUF):
                j = g * _NBUF + b
                wait_gather(j, b)
                scatter(j, b)
                start_gather(j + _NBUF, b)
            return carry

        lax.fori_loop(0, nch // _NBUF - 1, group, 0)

        for b in range(_NBUF):
            j = (nch - _NBUF) + b
            wait_gather(j, b)
            scatter(j, b)

    return k(idx, table)


def kernel(x, table):
    b_total = x.size
    assert b_total % (_NW * _CH) == 0
    nch = b_total // (_NW * _CH)
    assert nch % _NBUF == 0
    scaled = _scale_table(table)
    idx = x.reshape(_NW, nch, _CH).astype(jnp.int32)
    out = _gather(idx, scaled, nch)
    return out.reshape(x.shape + (D_EMB,))


# speedup vs baseline: 9.0398x; 1.1349x over previous
import functools
import math

import jax
import jax.numpy as jnp
from jax import lax
from jax.experimental import pallas as pl
from jax.experimental.pallas import tpu as pltpu
from jax.experimental.pallas import tpu_sc as plsc

D_EMB = 128
SCALE = math.sqrt(float(D_EMB))

_info = plsc.get_sparse_core_info()
_NC = _info.num_cores
_NS = _info.num_subcores
_NW = _NC * _NS

_CH = 128
_NBUF = 4


@functools.partial(jax.jit, static_argnames=("nch",))
def _gather(idx, table, nch):
    b_total = _NW * nch * _CH
    mesh = plsc.VectorSubcoreMesh(core_axis_name="c", subcore_axis_name="s")

    @functools.partial(
        pl.kernel,
        mesh=mesh,
        out_type=jax.ShapeDtypeStruct((b_total, D_EMB), jnp.float32),
        scratch_types=[
            pltpu.VMEM((nch, _CH), jnp.int32),
            pltpu.VMEM((_NBUF, _CH, D_EMB), jnp.float32),
            pltpu.SemaphoreType.DMA((_NBUF,)),
            pltpu.SemaphoreType.DMA((_NBUF,)),
        ],
    )
    def k(idx_hbm, table_hbm, out_hbm, idx_v, rows_v, gsem, ssem):
        wid = lax.axis_index("s") * _NC + lax.axis_index("c")
        base = wid * (nch * _CH)
        pltpu.sync_copy(idx_hbm.at[wid], idx_v)

        def start_gather(j, b):
            pltpu.async_copy(table_hbm.at[idx_v.at[j]], rows_v.at[b], gsem.at[b])

        def wait_gather(j, b):
            pltpu.make_async_copy(
                table_hbm.at[idx_v.at[j]], rows_v.at[b], gsem.at[b]
            ).wait()

        def start_scatter(j, b):
            pltpu.async_copy(
                rows_v.at[b], out_hbm.at[pl.ds(base + j * _CH, _CH)], ssem.at[b]
            )

        def wait_scatter(j, b):
            pltpu.make_async_copy(
                rows_v.at[b], out_hbm.at[pl.ds(base + j * _CH, _CH)], ssem.at[b]
            ).wait()

        def scale_buf(b):
            buf = rows_v.at[b]

            def body(r, carry):
                for u in range(2):
                    for c in range(8):
                        sl = (r * 2 + u, pl.ds(c * 16, 16))
                        buf[sl] = buf[sl] * SCALE
                return carry

            lax.fori_loop(0, _CH // 2, body, 0)

        def full_step(m, b, issue_gather):
            bp = (b - 1) % _NBUF
            wait_scatter(m - 1, bp)
            if issue_gather:
                start_gather(m - 1 + _NBUF, bp)
            wait_gather(m, b)
            scale_buf(b)
            start_scatter(m, b)

        for b in range(_NBUF):
            start_gather(b, b)

        wait_gather(0, 0)
        scale_buf(0)
        start_scatter(0, 0)
        for b in range(1, _NBUF):
            full_step(b, b, True)

        def group(g, carry):
            for b in range(_NBUF):
                full_step(g * _NBUF + b, b, True)
            return carry

        lax.fori_loop(1, nch // _NBUF - 1, group, 0)

        gl = nch // _NBUF - 1
        full_step(gl * _NBUF, 0, True)
        for b in range(1, _NBUF):
            full_step(gl * _NBUF + b, b, False)
        wait_scatter(nch - 1, _NBUF - 1)

    return k(idx, table)


def kernel(x, table):
    b_total = x.size
    assert b_total % (_NW * _CH) == 0
    nch = b_total // (_NW * _CH)
    assert nch % _NBUF == 0
    idx = x.reshape(_NW, nch, _CH).astype(jnp.int32)
    out = _gather(idx, table, nch)
    return out.reshape(x.shape + (D_EMB,))


# speedup vs baseline: 9.1927x; 1.0169x over previous
import functools
import math

import jax
import jax.numpy as jnp
from jax import lax
from jax.experimental import pallas as pl
from jax.experimental.pallas import tpu as pltpu
from jax.experimental.pallas import tpu_sc as plsc

D_EMB = 128
SCALE = math.sqrt(float(D_EMB))

_info = plsc.get_sparse_core_info()
_NC = _info.num_cores
_NS = _info.num_subcores
_NW = _NC * _NS

_CH = 128
_NBUF = 5
_K = 2


@functools.partial(jax.jit, static_argnames=("nch",))
def _gather(idx, table, nch):
    b_total = _NW * nch * _CH
    mesh = plsc.VectorSubcoreMesh(core_axis_name="c", subcore_axis_name="s")

    @functools.partial(
        pl.kernel,
        mesh=mesh,
        out_type=jax.ShapeDtypeStruct((b_total, D_EMB), jnp.float32),
        scratch_types=[
            pltpu.VMEM((nch, _CH), jnp.int32),
            pltpu.VMEM((_NBUF, _CH, D_EMB), jnp.float32),
            pltpu.SemaphoreType.DMA((_NBUF,)),
            pltpu.SemaphoreType.DMA((_NBUF,)),
        ],
    )
    def k(idx_hbm, table_hbm, out_hbm, idx_v, rows_v, gsem, ssem):
        wid = lax.axis_index("s") * _NC + lax.axis_index("c")
        base = wid * (nch * _CH)
        pltpu.sync_copy(idx_hbm.at[wid], idx_v)

        def start_gather(j, b):
            pltpu.async_copy(table_hbm.at[idx_v.at[j]], rows_v.at[b], gsem.at[b])

        def wait_gather(j, b):
            pltpu.make_async_copy(
                table_hbm.at[idx_v.at[j]], rows_v.at[b], gsem.at[b]
            ).wait()

        def start_scatter(j, b):
            pltpu.async_copy(
                rows_v.at[b], out_hbm.at[pl.ds(base + j * _CH, _CH)], ssem.at[b]
            )

        def wait_scatter(j, b):
            pltpu.make_async_copy(
                rows_v.at[b], out_hbm.at[pl.ds(base + j * _CH, _CH)], ssem.at[b]
            ).wait()

        def scale_buf(b):
            buf = rows_v.at[b]

            def body(r, carry):
                for u in range(2):
                    for c in range(8):
                        sl = (r * 2 + u, pl.ds(c * 16, 16))
                        buf[sl] = buf[sl] * SCALE
                return carry

            lax.fori_loop(0, _CH // 2, body, 0)

        def full_step(m, b, issue_gather):
            bp = (b - _K) % _NBUF
            wait_scatter(m - _K, bp)
            if issue_gather:
                start_gather(m - _K + _NBUF, bp)
            wait_gather(m, b)
            scale_buf(b)
            start_scatter(m, b)

        for b in range(_NBUF):
            start_gather(b, b)

        for b in range(_K):
            wait_gather(b, b)
            scale_buf(b)
            start_scatter(b, b)
        for b in range(_K, _NBUF):
            full_step(b, b, True)

        def group(g, carry):
            for b in range(_NBUF):
                full_step(g * _NBUF + b, b, True)
            return carry

        lax.fori_loop(1, nch // _NBUF - 1, group, 0)

        gl = nch // _NBUF - 1
        for b in range(_NBUF):
            m = gl * _NBUF + b
            full_step(m, b, m - _K + _NBUF < nch)
        for m in range(nch - _K, nch):
            wait_scatter(m, m % _NBUF)

    return k(idx, table)


def kernel(x, table):
    b_total = x.size
    assert b_total % (_NW * _CH) == 0
    nch = b_total // (_NW * _CH)
    assert nch % _NBUF == 0
    idx = x.reshape(_NW, nch, _CH).astype(jnp.int32)
    out = _gather(idx, table, nch)
    return out.reshape(x.shape + (D_EMB,))
